# baseline (speedup 1.0000x reference)
import functools
import math

import jax
import jax.numpy as jnp
from jax import lax
from jax.experimental import pallas as pl
from jax.experimental.pallas import tpu as pltpu
from jax.experimental.pallas import tpu_sc as plsc

_EMB = 32
_SCALE = math.sqrt(float(_EMB))
_NC = 2
_NS = 16
_NW = _NC * _NS
_CHUNK = 128
_NBG = 8
_NBS = 2
_NIB = 16


def _gather_kernel(n, table_hbm, idx_hbm, out_hbm,
                   ibuf, gbuf, sbuf, isem, gsem, ssem):
    wid = lax.axis_index("s") * _NC + lax.axis_index("c")
    grp0 = wid * n

    def start_idx(j):
        pltpu.async_copy(idx_hbm.at[grp0 + j], ibuf.at[j % _NIB],
                         isem.at[j % _NIB])

    def wait_idx(j):
        pltpu.make_async_copy(idx_hbm.at[grp0 + j], ibuf.at[j % _NIB],
                              isem.at[j % _NIB]).wait()

    def start_gather(j, bg):
        pltpu.async_copy(table_hbm.at[ibuf.at[j % _NIB]], gbuf.at[bg],
                         gsem.at[bg])

    def wait_gather(j, bg):
        pltpu.make_async_copy(table_hbm.at[ibuf.at[j % _NIB]], gbuf.at[bg],
                              gsem.at[bg]).wait()

    def start_store(j, bs):
        pltpu.async_copy(sbuf.at[bs], out_hbm.at[grp0 + j], ssem.at[bs])

    def wait_store(j, bs):
        pltpu.make_async_copy(sbuf.at[bs], out_hbm.at[grp0 + j],
                              ssem.at[bs]).wait()

    for j in range(_NIB):
        start_idx(j)
    for j in range(_NBG):
        wait_idx(j)
        start_gather(j, j)

    def outer(g, c):
        for b in range(_NBG):
            j = g * _NBG + b
            wait_gather(j, b)

            @pl.when(j >= _NBS)
            def _():
                wait_store(j - _NBS, b % _NBS)

            sbuf[b % _NBS] = gbuf[b] * _SCALE
            start_store(j, b % _NBS)

            @pl.when(j + _NBG < n)
            def _():
                wait_idx(j + _NBG)
                start_gather(j + _NBG, b)

            @pl.when(j + _NIB < n)
            def _():
                start_idx(j + _NIB)
        return c
    lax.fori_loop(0, n // _NBG, outer, 0)

    for j in range(n - _NBS, n):
        wait_store(j, j % _NBS)


@functools.partial(jax.jit, static_argnums=(2,))
def _embed(idx, table, n):
    mesh = plsc.VectorSubcoreMesh(
        core_axis_name="c", subcore_axis_name="s",
        num_cores=_NC, num_subcores=_NS)
    total = idx.shape[0]
    run = pl.kernel(
        functools.partial(_gather_kernel, n),
        out_type=jax.ShapeDtypeStruct((total, _CHUNK, _EMB), jnp.float32),
        mesh=mesh,
        scratch_types=[
            pltpu.VMEM((_NIB, _CHUNK), jnp.int32),
            pltpu.VMEM((_NBG, _CHUNK, _EMB), jnp.float32),
            pltpu.VMEM((_NBS, _CHUNK, _EMB), jnp.float32),
            pltpu.SemaphoreType.DMA((_NIB,)),
            pltpu.SemaphoreType.DMA((_NBG,)),
            pltpu.SemaphoreType.DMA((_NBS,)),
        ],
        compiler_params=pltpu.CompilerParams(use_tc_tiling_on_sc=False),
    )
    return run(table, idx)


def kernel(tokens, table):
    b = tokens.size
    assert b % (_NW * _CHUNK) == 0
    n = b // (_NW * _CHUNK)
    assert n % _NBG == 0 and _NBG % _NBS == 0 and n > _NIB + _NBG
    idx = tokens.reshape(_NW * n, _CHUNK).astype(jnp.int32)
    out = _embed(idx, table, n)
    return out.reshape(*tokens.shape, _EMB)


# speedup vs baseline: 1.0176x; 1.0176x over previous
import functools
import math

import jax
import jax.numpy as jnp
from jax import lax
from jax.experimental import pallas as pl
from jax.experimental.pallas import tpu as pltpu
from jax.experimental.pallas import tpu_sc as plsc

_EMB = 32
_SCALE = math.sqrt(float(_EMB))
_NC = 2
_NS = 16
_NW = _NC * _NS
_CHUNK = 256
_NBG = 4
_NBS = 2
_NIB = 8


def _gather_kernel(n, table_hbm, idx_hbm, out_hbm,
                   ibuf, gbuf, sbuf, isem, gsem, ssem):
    wid = lax.axis_index("s") * _NC + lax.axis_index("c")
    grp0 = wid * n

    def start_idx(j):
        pltpu.async_copy(idx_hbm.at[grp0 + j], ibuf.at[j % _NIB],
                         isem.at[j % _NIB])

    def wait_idx(j):
        pltpu.make_async_copy(idx_hbm.at[grp0 + j], ibuf.at[j % _NIB],
                              isem.at[j % _NIB]).wait()

    def start_gather(j, bg):
        pltpu.async_copy(table_hbm.at[ibuf.at[j % _NIB]], gbuf.at[bg],
                         gsem.at[bg])

    def wait_gather(j, bg):
        pltpu.make_async_copy(table_hbm.at[ibuf.at[j % _NIB]], gbuf.at[bg],
                              gsem.at[bg]).wait()

    def start_store(j, bs):
        pltpu.async_copy(sbuf.at[bs], out_hbm.at[grp0 + j], ssem.at[bs])

    def wait_store(j, bs):
        pltpu.make_async_copy(sbuf.at[bs], out_hbm.at[grp0 + j],
                              ssem.at[bs]).wait()

    for j in range(_NIB):
        start_idx(j)
    for j in range(_NBG):
        wait_idx(j)
        start_gather(j, j)

    def outer(g, c):
        for b in range(_NBG):
            j = g * _NBG + b
            wait_gather(j, b)

            @pl.when(j >= _NBS)
            def _():
                wait_store(j - _NBS, b % _NBS)

            for r in range(0, _CHUNK, 16):
                sbuf[b % _NBS, pl.ds(r, 16)] = (
                    gbuf[b, pl.ds(r, 16)] * _SCALE)
            start_store(j, b % _NBS)

            @pl.when(j + _NBG < n)
            def _():
                wait_idx(j + _NBG)
                start_gather(j + _NBG, b)

            @pl.when(j + _NIB < n)
            def _():
                start_idx(j + _NIB)
        return c
    lax.fori_loop(0, n // _NBG, outer, 0)

    for j in range(n - _NBS, n):
        wait_store(j, j % _NBS)


@functools.partial(jax.jit, static_argnums=(2,))
def _embed(idx, table, n):
    mesh = plsc.VectorSubcoreMesh(
        core_axis_name="c", subcore_axis_name="s",
        num_cores=_NC, num_subcores=_NS)
    total = idx.shape[0]
    run = pl.kernel(
        functools.partial(_gather_kernel, n),
        out_type=jax.ShapeDtypeStruct((total, _CHUNK, _EMB), jnp.float32),
        mesh=mesh,
        scratch_types=[
            pltpu.VMEM((_NIB, _CHUNK), jnp.int32),
            pltpu.VMEM((_NBG, _CHUNK, _EMB), jnp.float32),
            pltpu.VMEM((_NBS, _CHUNK, _EMB), jnp.float32),
            pltpu.SemaphoreType.DMA((_NIB,)),
            pltpu.SemaphoreType.DMA((_NBG,)),
            pltpu.SemaphoreType.DMA((_NBS,)),
        ],
        compiler_params=pltpu.CompilerParams(use_tc_tiling_on_sc=False),
    )
    return run(table, idx)


def kernel(tokens, table):
    b = tokens.size
    assert b % (_NW * _CHUNK) == 0
    n = b // (_NW * _CHUNK)
    assert n % _NBG == 0 and _NBG % _NBS == 0 and n > _NIB + _NBG
    idx = tokens.reshape(_NW * n, _CHUNK).astype(jnp.int32)
    out = _embed(idx, table, n)
    return out.reshape(*tokens.shape, _EMB)


# speedup vs baseline: 1.0223x; 1.0046x over previous
import functools
import math

import jax
import jax.numpy as jnp
from jax import lax
from jax.experimental import pallas as pl
from jax.experimental.pallas import tpu as pltpu
from jax.experimental.pallas import tpu_sc as plsc

_EMB = 32
_SCALE = math.sqrt(float(_EMB))
_NC = 2
_NS = 16
_NW = _NC * _NS
_CHUNK = 128
_NBG = 4
_NBS = 2
_NIB = 8


def _gather_kernel(n, table_hbm, idx_hbm, out_hbm,
                   ibuf, gbuf, sbuf, isem, gsem, ssem):
    wid = lax.axis_index("s") * _NC + lax.axis_index("c")
    grp0 = wid * n

    def start_idx(j):
        pltpu.async_copy(idx_hbm.at[grp0 + j], ibuf.at[j % _NIB],
                         isem.at[j % _NIB])

    def wait_idx(j):
        pltpu.make_async_copy(idx_hbm.at[grp0 + j], ibuf.at[j % _NIB],
                              isem.at[j % _NIB]).wait()

    def start_gather(j, bg):
        pltpu.async_copy(table_hbm.at[ibuf.at[j % _NIB]], gbuf.at[bg],
                         gsem.at[bg])

    def wait_gather(j, bg):
        pltpu.make_async_copy(table_hbm.at[ibuf.at[j % _NIB]], gbuf.at[bg],
                              gsem.at[bg]).wait()

    def start_store(j, bs):
        pltpu.async_copy(sbuf.at[bs], out_hbm.at[grp0 + j], ssem.at[bs])

    def wait_store(j, bs):
        pltpu.make_async_copy(sbuf.at[bs], out_hbm.at[grp0 + j],
                              ssem.at[bs]).wait()

    def body(j, bg, bs, with_store_wait=True, with_gather=True,
             with_idx=True):
        wait_gather(j, bg)
        if with_store_wait:
            wait_store(j - _NBS, bs)
        sbuf[bs] = gbuf[bg] * _SCALE
        start_store(j, bs)
        if with_gather:
            wait_idx(j + _NBG)
            start_gather(j + _NBG, bg)
        if with_idx:
            start_idx(j + _NIB)

    for j in range(_NIB):
        start_idx(j)
    for j in range(_NBG):
        wait_idx(j)
        start_gather(j, j)
    for j in range(_NBG):
        body(j, j % _NBG, j % _NBS, with_store_wait=(j >= _NBS))

    def outer(g, c):
        for b in range(_NBG):
            j = g * _NBG + b
            body(j, b, b % _NBS)
        return c
    lax.fori_loop(1, (n - _NIB) // _NBG, outer, 0)

    for j in range(n - _NIB, n - _NBG):
        body(j, j % _NBG, j % _NBS, with_idx=False)
    for j in range(n - _NBG, n):
        body(j, j % _NBG, j % _NBS, with_gather=False, with_idx=False)
    for j in range(n - _NBS, n):
        wait_store(j, j % _NBS)


@functools.partial(jax.jit, static_argnums=(2,))
def _embed(idx, table, n):
    mesh = plsc.VectorSubcoreMesh(
        core_axis_name="c", subcore_axis_name="s",
        num_cores=_NC, num_subcores=_NS)
    total = idx.shape[0]
    run = pl.kernel(
        functools.partial(_gather_kernel, n),
        out_type=jax.ShapeDtypeStruct((total, _CHUNK, _EMB), jnp.float32),
        mesh=mesh,
        scratch_types=[
            pltpu.VMEM((_NIB, _CHUNK), jnp.int32),
            pltpu.VMEM((_NBG, _CHUNK, _EMB), jnp.float32),
            pltpu.VMEM((_NBS, _CHUNK, _EMB), jnp.float32),
            pltpu.SemaphoreType.DMA((_NIB,)),
            pltpu.SemaphoreType.DMA((_NBG,)),
            pltpu.SemaphoreType.DMA((_NBS,)),
        ],
        compiler_params=pltpu.CompilerParams(use_tc_tiling_on_sc=False),
    )
    return run(table, idx)


def kernel(tokens, table):
    b = tokens.size
    assert b % (_NW * _CHUNK) == 0
    n = b // (_NW * _CHUNK)
    assert (n - _NIB) % _NBG == 0 and _NBG % _NBS == 0 and n > _NIB + _NBG
    idx = tokens.reshape(_NW * n, _CHUNK).astype(jnp.int32)
    out = _embed(idx, table, n)
    return out.reshape(*tokens.shape, _EMB)
